# baseline (speedup 1.0000x reference)
import functools

import jax
import jax.numpy as jnp
from jax import lax
from jax.experimental import pallas as pl
from jax.experimental.pallas import tpu as pltpu
from jax.experimental.pallas import tpu_sc as plsc


def _prep_body(k_nb, pct_ref, p_ref):
    x = pct_ref[0]
    n = x.shape[-1]
    x = x.reshape(k_nb, 3, n)
    xyz = jnp.mean(x, axis=0)
    sq = jnp.sum(xyz * xyz, axis=0, keepdims=True)
    p_ref[0] = jnp.concatenate([xyz, sq], axis=0)


def _prep(pct, interpret=False):
    b, threek, n = pct.shape
    return pl.pallas_call(
        functools.partial(_prep_body, threek // 3),
        grid=(b,),
        in_specs=[pl.BlockSpec((1, threek, n), lambda i: (i, 0, 0))],
        out_specs=pl.BlockSpec((1, 4, n), lambda i: (i, 0, 0)),
        out_shape=jax.ShapeDtypeStruct((b, 4, n), jnp.float32),
        interpret=interpret,
    )(pct)


_ROUNDS = 3
_LANES = 128


def _knn_body(rb, k_nb, pfull_ref, prows_ref, idx_ref, d_ref, cv_ref, ci_ref):
    bi = pl.program_id(0)
    n = pfull_ref.shape[-1]
    nsl = n // _LANES
    cw = cv_ref.shape[-1]
    inf = jnp.float32(jnp.inf)
    xyzT = pfull_ref[0, 0:3, :]
    sqj = pfull_ref[0, 3:4, :]
    lhs = prows_ref[0, 0:3, :]
    sqi = prows_ref[0, 3:4, :]
    dn = (((0,), (0,)), ((), ()))
    dot = lax.dot_general(lhs, xyzT, dn, preferred_element_type=jnp.float32)
    ones = jnp.ones((1, n), jnp.float32)
    sqib = lax.dot_general(sqi, ones, dn, preferred_element_type=jnp.float32)
    d_ref[...] = (sqib + sqj) - 2.0 * dot
    cv_ref[...] = jnp.full((rb, cw), inf, jnp.float32)
    ci_ref[...] = jnp.full((rb, cw), jnp.int32(1 << 30), jnp.int32)

    lane = lax.broadcasted_iota(jnp.int32, (rb, _LANES), 1)

    def fold_min_argmin():
        u = d_ref[:, 0:_LANES]
        sidx = jnp.zeros((rb, _LANES), jnp.int32)
        for s in range(1, nsl):
            sl = d_ref[:, s * _LANES : (s + 1) * _LANES]
            take = sl < u
            u = jnp.where(take, sl, u)
            sidx = jnp.where(take, s, sidx)
        return u, sidx

    for r in range(_ROUNDS):
        u, sidx = fold_min_argmin()
        cv_ref[:, r * _LANES : (r + 1) * _LANES] = u
        ci_ref[:, r * _LANES : (r + 1) * _LANES] = sidx * _LANES + lane
        for s in range(nsl):
            sl = d_ref[:, s * _LANES : (s + 1) * _LANES]
            d_ref[:, s * _LANES : (s + 1) * _LANES] = jnp.where(
                sidx == s, inf, sl)

    u = d_ref[:, 0:_LANES]
    for s in range(1, nsl):
        u = jnp.minimum(u, d_ref[:, s * _LANES : (s + 1) * _LANES])
    tau = jnp.min(u, axis=1, keepdims=True)
    cnt = jnp.sum((cv_ref[...] < tau).astype(jnp.int32), axis=1,
                  keepdims=True)
    ok = jnp.min(cnt) >= k_nb

    @pl.when(jnp.logical_not(ok))
    def _fallback():
        iota = lax.broadcasted_iota(jnp.int32, (rb, n), 1)
        bign = jnp.int32(n)
        base = _ROUNDS * _LANES
        for k in range(k_nb):
            dd = d_ref[...]
            m = jnp.min(dd, axis=1, keepdims=True)
            t = jnp.where(dd == m, iota, bign)
            a = jnp.min(t, axis=1, keepdims=True)
            cv_ref[:, base + k : base + k + 1] = m
            ci_ref[:, base + k : base + k + 1] = a
            d_ref[...] = jnp.where(iota == a, inf, dd)

    cv = cv_ref[...]
    ci = ci_ref[...]
    bigi = jnp.int32(1 << 30)
    iota_k = lax.broadcasted_iota(jnp.int32, (rb, k_nb), 1)
    res = jnp.zeros((rb, k_nb), jnp.int32)
    for k in range(k_nb):
        m = jnp.min(cv, axis=1, keepdims=True)
        t = jnp.where(cv == m, ci, bigi)
        a = jnp.min(t, axis=1, keepdims=True)
        res = jnp.where(iota_k == k, a, res)
        cv = jnp.where(ci == a, inf, cv)
    idx_ref[0] = res + bi * n


def _knn(p, k_nb, rb, interpret=False):
    b, _, n = p.shape
    cw = (_ROUNDS + 1) * _LANES
    return pl.pallas_call(
        functools.partial(_knn_body, rb, k_nb),
        grid=(b, n // rb),
        in_specs=[
            pl.BlockSpec((1, 4, n), lambda i, j: (i, 0, 0)),
            pl.BlockSpec((1, 4, rb), lambda i, j: (i, 0, j)),
        ],
        out_specs=pl.BlockSpec((1, rb, k_nb), lambda i, j: (i, j, 0)),
        out_shape=jax.ShapeDtypeStruct((b, n, k_nb), jnp.int32),
        scratch_shapes=[
            pltpu.VMEM((rb, n), jnp.float32),
            pltpu.VMEM((rb, cw), jnp.float32),
            pltpu.VMEM((rb, cw), jnp.int32),
        ],
        interpret=interpret,
    )(p, p)


def _sc_gather(table, idx2d):
    info = plsc.get_sparse_core_info()
    nw = info.num_cores * info.num_subcores
    nrows_idx, lanes = idx2d.shape
    jpw = nrows_idx // nw
    cin = table.shape[-1]
    m_rows = nrows_idx * lanes
    mesh = plsc.VectorSubcoreMesh(core_axis_name="c", subcore_axis_name="s")

    @functools.partial(
        pl.kernel,
        mesh=mesh,
        compiler_params=pltpu.CompilerParams(use_tc_tiling_on_sc=False),
        out_type=jax.ShapeDtypeStruct((m_rows, cin), jnp.float32),
        scratch_types=[
            pltpu.VMEM((jpw, lanes), jnp.int32),
            pltpu.VMEM((lanes, cin), jnp.float32),
            pltpu.VMEM((lanes, cin), jnp.float32),
            pltpu.SemaphoreType.DMA,
            pltpu.SemaphoreType.DMA,
        ],
    )
    def k(table_hbm, idx_hbm, out_hbm, idx_all, rows0, rows1, sem0, sem1):
        wid = lax.axis_index("s") * info.num_cores + lax.axis_index("c")
        base = wid * jpw
        pltpu.sync_copy(idx_hbm.at[pl.ds(base, jpw)], idx_all)
        rows = (rows0, rows1)
        sems = (sem0, sem1)

        pltpu.async_copy(table_hbm.at[idx_all.at[0]], rows0, sem0)
        pltpu.async_copy(table_hbm.at[idx_all.at[1]], rows1, sem1)

        def body(i, _):
            for t in range(2):
                j = 2 * i + t
                pltpu.make_async_copy(
                    table_hbm.at[idx_all.at[j]], rows[t], sems[t]
                ).wait()
                pltpu.sync_copy(
                    rows[t], out_hbm.at[pl.ds((base + j) * lanes, lanes)]
                )

                @pl.when(j + 2 < jpw)
                def _():
                    pltpu.async_copy(
                        table_hbm.at[idx_all.at[j + 2]], rows[t], sems[t]
                    )

            return 0

        lax.fori_loop(0, jpw // 2, body, 0)

    return k(table, idx2d)


def _mlp1_body(m_rows, fgw_ref, pc_ref, w1f_ref, w1p_ref, b1_ref,
               y1_ref, st_ref):
    i = pl.program_id(0)
    y = (
        jnp.dot(fgw_ref[...], w1f_ref[...], preferred_element_type=jnp.float32)
        + jnp.dot(pc_ref[...], w1p_ref[...], preferred_element_type=jnp.float32)
        + b1_ref[...]
    )
    y1_ref[...] = y

    @pl.when(i == 0)
    def _():
        st_ref[...] = jnp.zeros_like(st_ref)

    st_ref[0:1, :] += jnp.sum(y, axis=0, keepdims=True)
    st_ref[1:2, :] += jnp.sum(y * y, axis=0, keepdims=True)


def _bn_affine(st_ref, g_ref, be_ref, m_rows, k_nb, cout):
    ssum = st_ref[0:1, 0:cout]
    ssq = st_ref[1:2, 0:cout]
    for k in range(1, k_nb):
        ssum = ssum + st_ref[0:1, k * cout : (k + 1) * cout]
        ssq = ssq + st_ref[1:2, k * cout : (k + 1) * cout]
    mm = ssum / m_rows
    vv = ssq / m_rows - mm * mm
    a = g_ref[...] / jnp.sqrt(vv + 1e-5)
    c = be_ref[...] - mm * a
    a = jnp.concatenate([a] * k_nb, axis=1)
    c = jnp.concatenate([c] * k_nb, axis=1)
    return a, c


def _mlp2_body(m_rows, k_nb, cout, y1_ref, st1_ref, w2_ref, b2_ref, g1_ref,
               be1_ref, y2_ref, st2_ref):
    i = pl.program_id(0)
    a, c = _bn_affine(st1_ref, g1_ref, be1_ref, m_rows, k_nb, cout)
    h = jnp.maximum(y1_ref[...] * a + c, 0.0)
    y = jnp.dot(h, w2_ref[...], preferred_element_type=jnp.float32) + b2_ref[...]
    y2_ref[...] = y

    @pl.when(i == 0)
    def _():
        st2_ref[...] = jnp.zeros_like(st2_ref)

    st2_ref[0:1, :] += jnp.sum(y, axis=0, keepdims=True)
    st2_ref[1:2, :] += jnp.sum(y * y, axis=0, keepdims=True)


def _mlp3_body(m_rows, k_nb, cout, y2_ref, st2_ref, w3_ref, b3_ref, g2_ref,
               be2_ref, out_ref):
    a, c = _bn_affine(st2_ref, g2_ref, be2_ref, m_rows, k_nb, cout)
    h = jnp.maximum(y2_ref[...] * a + c, 0.0)
    y = jnp.dot(h, w3_ref[...], preferred_element_type=jnp.float32) + b3_ref[...]
    r = y[:, 0:cout]
    for k in range(1, k_nb):
        r = jnp.maximum(r, y[:, k * cout : (k + 1) * cout])
    out_ref[...] = r


def _mlp(fgw, pc48, k_nb, W1, b1, g1, be1, W2, b2, g2, be2, W3, b3, rn,
         interpret=False):
    npts, wide = fgw.shape
    cout = W1.shape[-1]
    cin = W1.shape[0] - 3
    m_rows = float(npts * k_nb)
    grid = (npts // rn,)
    row2 = lambda i: (i, 0)
    whole = lambda i: (0, 0)
    eye = jnp.eye(k_nb, dtype=jnp.float32)
    w1f = jnp.kron(eye, W1[0:cin, :])
    w1p = jnp.kron(eye, W1[cin : cin + 3, :])
    w2w = jnp.kron(eye, W2)
    w3w = jnp.kron(eye, W3)
    bt = lambda v: jnp.tile(v.reshape(1, -1), (1, k_nb))
    vec = lambda v: v.reshape(1, -1)
    st_shape = jax.ShapeDtypeStruct((8, wide), jnp.float32)
    st_spec = pl.BlockSpec((8, wide), whole)

    y1, st1 = pl.pallas_call(
        functools.partial(_mlp1_body, m_rows),
        grid=grid,
        in_specs=[
            pl.BlockSpec((rn, wide), row2),
            pl.BlockSpec((rn, 3 * k_nb), row2),
            pl.BlockSpec(w1f.shape, whole),
            pl.BlockSpec(w1p.shape, whole),
            pl.BlockSpec((1, wide), whole),
        ],
        out_specs=[pl.BlockSpec((rn, wide), row2), st_spec],
        out_shape=[jax.ShapeDtypeStruct((npts, wide), jnp.float32), st_shape],
        interpret=interpret,
    )(fgw, pc48, w1f, w1p, bt(b1))

    y2, st2 = pl.pallas_call(
        functools.partial(_mlp2_body, m_rows, k_nb, cout),
        grid=grid,
        in_specs=[
            pl.BlockSpec((rn, wide), row2),
            st_spec,
            pl.BlockSpec(w2w.shape, whole),
            pl.BlockSpec((1, wide), whole),
            pl.BlockSpec((1, cout), whole),
            pl.BlockSpec((1, cout), whole),
        ],
        out_specs=[pl.BlockSpec((rn, wide), row2), st_spec],
        out_shape=[jax.ShapeDtypeStruct((npts, wide), jnp.float32), st_shape],
        interpret=interpret,
    )(y1, st1, w2w, bt(b2), vec(g1), vec(be1))

    out = pl.pallas_call(
        functools.partial(_mlp3_body, m_rows, k_nb, cout),
        grid=grid,
        in_specs=[
            pl.BlockSpec((rn, wide), row2),
            st_spec,
            pl.BlockSpec(w3w.shape, whole),
            pl.BlockSpec((1, wide), whole),
            pl.BlockSpec((1, cout), whole),
            pl.BlockSpec((1, cout), whole),
        ],
        out_specs=pl.BlockSpec((rn, cout), row2),
        out_shape=jax.ShapeDtypeStruct((npts, cout), jnp.float32),
        interpret=interpret,
    )(y2, st2, w3w, bt(b3), vec(g2), vec(be2))
    return out


def kernel(features, position_condition, W1, b1, g1, be1, W2, b2, g2, be2,
           W3, b3):
    b, n, cin = features.shape
    k_nb = position_condition.shape[2]
    m_rows = b * n * k_nb

    pc48 = position_condition.reshape(b, n, 3 * k_nb)
    pct = position_condition.transpose(0, 2, 3, 1).reshape(b, 3 * k_nb, n)
    p = _prep(pct)
    idx = _knn(p, k_nb, rb=512)
    table = features.reshape(b * n, cin)
    fg = _sc_gather(table, idx.reshape(m_rows // 128, 128))
    fgw = fg.reshape(b * n, k_nb * cin)
    out = _mlp(fgw, pc48.reshape(b * n, 3 * k_nb), k_nb,
               W1, b1, g1, be1, W2, b2, g2, be2, W3, b3, rn=512)
    return out.reshape(b, n, W3.shape[-1])


# speedup vs baseline: 1.1351x; 1.1351x over previous
import functools

import jax
import jax.numpy as jnp
from jax import lax
from jax.experimental import pallas as pl
from jax.experimental.pallas import tpu as pltpu
from jax.experimental.pallas import tpu_sc as plsc


def _prep_body(k_nb, pct_ref, p_ref):
    x = pct_ref[0]
    n = x.shape[-1]
    x = x.reshape(k_nb, 3, n)
    xyz = jnp.mean(x, axis=0)
    sq = jnp.sum(xyz * xyz, axis=0, keepdims=True)
    p_ref[0] = jnp.concatenate([xyz, sq], axis=0)


def _prep(pct, interpret=False):
    b, threek, n = pct.shape
    return pl.pallas_call(
        functools.partial(_prep_body, threek // 3),
        grid=(b,),
        in_specs=[pl.BlockSpec((1, threek, n), lambda i: (i, 0, 0))],
        out_specs=pl.BlockSpec((1, 4, n), lambda i: (i, 0, 0)),
        out_shape=jax.ShapeDtypeStruct((b, 4, n), jnp.float32),
        interpret=interpret,
    )(pct)


_ROUNDS = 3
_LANES = 128


def _knn_body(rb, k_nb, pfull_ref, prows_ref, idx_ref, d_ref, cv_ref, ci_ref):
    bi = pl.program_id(0)
    n = pfull_ref.shape[-1]
    nsl = n // _LANES
    cw = cv_ref.shape[-1]
    inf = jnp.float32(jnp.inf)
    xyzT = pfull_ref[0, 0:3, :]
    sqj = pfull_ref[0, 3:4, :]
    lhs = prows_ref[0, 0:3, :]
    sqi = prows_ref[0, 3:4, :]
    dn = (((0,), (0,)), ((), ()))
    dot = lax.dot_general(lhs, xyzT, dn, preferred_element_type=jnp.float32)
    ones = jnp.ones((1, n), jnp.float32)
    sqib = lax.dot_general(sqi, ones, dn, preferred_element_type=jnp.float32)
    d_ref[...] = (sqib + sqj) - 2.0 * dot
    cv_ref[...] = jnp.full((rb, cw), inf, jnp.float32)
    ci_ref[...] = jnp.full((rb, cw), jnp.int32(1 << 30), jnp.int32)

    lane = lax.broadcasted_iota(jnp.int32, (rb, _LANES), 1)

    def fold_min_argmin():
        u = d_ref[:, 0:_LANES]
        sidx = jnp.zeros((rb, _LANES), jnp.int32)
        for s in range(1, nsl):
            sl = d_ref[:, s * _LANES : (s + 1) * _LANES]
            take = sl < u
            u = jnp.where(take, sl, u)
            sidx = jnp.where(take, s, sidx)
        return u, sidx

    for r in range(_ROUNDS):
        u, sidx = fold_min_argmin()
        cv_ref[:, r * _LANES : (r + 1) * _LANES] = u
        ci_ref[:, r * _LANES : (r + 1) * _LANES] = sidx * _LANES + lane
        for s in range(nsl):
            sl = d_ref[:, s * _LANES : (s + 1) * _LANES]
            d_ref[:, s * _LANES : (s + 1) * _LANES] = jnp.where(
                sidx == s, inf, sl)

    u = d_ref[:, 0:_LANES]
    for s in range(1, nsl):
        u = jnp.minimum(u, d_ref[:, s * _LANES : (s + 1) * _LANES])
    tau = jnp.min(u, axis=1, keepdims=True)
    cnt = jnp.sum((cv_ref[...] < tau).astype(jnp.int32), axis=1,
                  keepdims=True)
    ok = jnp.min(cnt) >= k_nb

    @pl.when(jnp.logical_not(ok))
    def _fallback():
        iota = lax.broadcasted_iota(jnp.int32, (rb, n), 1)
        bign = jnp.int32(n)
        base = _ROUNDS * _LANES
        for k in range(k_nb):
            dd = d_ref[...]
            m = jnp.min(dd, axis=1, keepdims=True)
            t = jnp.where(dd == m, iota, bign)
            a = jnp.min(t, axis=1, keepdims=True)
            cv_ref[:, base + k : base + k + 1] = m
            ci_ref[:, base + k : base + k + 1] = a
            d_ref[...] = jnp.where(iota == a, inf, dd)

    cv = cv_ref[...]
    ci = ci_ref[...]
    bigi = jnp.int32(1 << 30)
    iota_k = lax.broadcasted_iota(jnp.int32, (rb, k_nb), 1)
    res = jnp.zeros((rb, k_nb), jnp.int32)
    for k in range(k_nb):
        m = jnp.min(cv, axis=1, keepdims=True)
        t = jnp.where(cv == m, ci, bigi)
        a = jnp.min(t, axis=1, keepdims=True)
        res = jnp.where(iota_k == k, a, res)
        cv = jnp.where(ci == a, inf, cv)
    idx_ref[0] = res + bi * n


def _knn(p, k_nb, rb, interpret=False):
    b, _, n = p.shape
    cw = (_ROUNDS + 1) * _LANES
    return pl.pallas_call(
        functools.partial(_knn_body, rb, k_nb),
        grid=(b, n // rb),
        in_specs=[
            pl.BlockSpec((1, 4, n), lambda i, j: (i, 0, 0)),
            pl.BlockSpec((1, 4, rb), lambda i, j: (i, 0, j)),
        ],
        out_specs=pl.BlockSpec((1, rb, k_nb), lambda i, j: (i, j, 0)),
        out_shape=jax.ShapeDtypeStruct((b, n, k_nb), jnp.int32),
        scratch_shapes=[
            pltpu.VMEM((rb, n), jnp.float32),
            pltpu.VMEM((rb, cw), jnp.float32),
            pltpu.VMEM((rb, cw), jnp.int32),
        ],
        interpret=interpret,
    )(p, p)


def _sc_gather(table, idx2d):
    info = plsc.get_sparse_core_info()
    nw = info.num_cores * info.num_subcores
    nrows_idx, lanes = idx2d.shape
    jpw = nrows_idx // nw
    cin = table.shape[-1]
    m_rows = nrows_idx * lanes
    mesh = plsc.VectorSubcoreMesh(core_axis_name="c", subcore_axis_name="s")

    @functools.partial(
        pl.kernel,
        mesh=mesh,
        compiler_params=pltpu.CompilerParams(use_tc_tiling_on_sc=False),
        out_type=jax.ShapeDtypeStruct((m_rows, cin), jnp.float32),
        scratch_types=[
            pltpu.VMEM((jpw, lanes), jnp.int32),
            pltpu.VMEM((lanes, cin), jnp.float32),
            pltpu.VMEM((lanes, cin), jnp.float32),
            pltpu.SemaphoreType.DMA,
            pltpu.SemaphoreType.DMA,
        ],
    )
    def k(table_hbm, idx_hbm, out_hbm, idx_all, rows0, rows1, sem0, sem1):
        wid = lax.axis_index("s") * info.num_cores + lax.axis_index("c")
        base = wid * jpw
        pltpu.sync_copy(idx_hbm.at[pl.ds(base, jpw)], idx_all)
        rows = (rows0, rows1)
        sems = (sem0, sem1)

        pltpu.async_copy(table_hbm.at[idx_all.at[0]], rows0, sem0)
        pltpu.async_copy(table_hbm.at[idx_all.at[1]], rows1, sem1)

        def body(i, _):
            for t in range(2):
                j = 2 * i + t
                pltpu.make_async_copy(
                    table_hbm.at[idx_all.at[j]], rows[t], sems[t]
                ).wait()
                pltpu.sync_copy(
                    rows[t], out_hbm.at[pl.ds((base + j) * lanes, lanes)]
                )

                @pl.when(j + 2 < jpw)
                def _():
                    pltpu.async_copy(
                        table_hbm.at[idx_all.at[j + 2]], rows[t], sems[t]
                    )

            return 0

        lax.fori_loop(0, jpw // 2, body, 0)

    return k(table, idx2d)


def _mlp1_body(m_rows, fgw_ref, pc_ref, w1f_ref, w1p_ref, b1_ref,
               y1_ref, st_ref):
    i = pl.program_id(0)
    y = (
        jnp.dot(fgw_ref[...], w1f_ref[...], preferred_element_type=jnp.float32)
        + jnp.dot(pc_ref[...], w1p_ref[...], preferred_element_type=jnp.float32)
        + b1_ref[...]
    )
    y1_ref[...] = y

    @pl.when(i == 0)
    def _():
        st_ref[...] = jnp.zeros_like(st_ref)

    st_ref[0:1, :] += jnp.sum(y, axis=0, keepdims=True)
    st_ref[1:2, :] += jnp.sum(y * y, axis=0, keepdims=True)


def _bn_affine(st_ref, g_ref, be_ref, m_rows, k_nb, cout):
    ssum = st_ref[0:1, 0:cout]
    ssq = st_ref[1:2, 0:cout]
    for k in range(1, k_nb):
        ssum = ssum + st_ref[0:1, k * cout : (k + 1) * cout]
        ssq = ssq + st_ref[1:2, k * cout : (k + 1) * cout]
    mm = ssum / m_rows
    vv = ssq / m_rows - mm * mm
    a = g_ref[...] / jnp.sqrt(vv + 1e-5)
    c = be_ref[...] - mm * a
    a = jnp.concatenate([a] * k_nb, axis=1)
    c = jnp.concatenate([c] * k_nb, axis=1)
    return a, c


def _mlp2_body(m_rows, k_nb, cout, y1_ref, st1_ref, w2_ref, b2_ref, g1_ref,
               be1_ref, y2_ref, st2_ref):
    i = pl.program_id(0)
    a, c = _bn_affine(st1_ref, g1_ref, be1_ref, m_rows, k_nb, cout)
    h = jnp.maximum(y1_ref[...] * a + c, 0.0)
    y = jnp.dot(h, w2_ref[...], preferred_element_type=jnp.float32) + b2_ref[...]
    y2_ref[...] = y

    @pl.when(i == 0)
    def _():
        st2_ref[...] = jnp.zeros_like(st2_ref)

    st2_ref[0:1, :] += jnp.sum(y, axis=0, keepdims=True)
    st2_ref[1:2, :] += jnp.sum(y * y, axis=0, keepdims=True)


def _mlp3_body(m_rows, k_nb, cout, y2_ref, st2_ref, w3_ref, b3_ref, g2_ref,
               be2_ref, out_ref):
    a, c = _bn_affine(st2_ref, g2_ref, be2_ref, m_rows, k_nb, cout)
    h = jnp.maximum(y2_ref[...] * a + c, 0.0)
    y = jnp.dot(h, w3_ref[...], preferred_element_type=jnp.float32) + b3_ref[...]
    r = y[:, 0:cout]
    for k in range(1, k_nb):
        r = jnp.maximum(r, y[:, k * cout : (k + 1) * cout])
    out_ref[...] = r


def _mlp(fgw, pc48, k_nb, W1, b1, g1, be1, W2, b2, g2, be2, W3, b3, rn,
         interpret=False):
    npts, wide = fgw.shape
    cout = W1.shape[-1]
    cin = W1.shape[0] - 3
    m_rows = float(npts * k_nb)
    grid = (npts // rn,)
    row2 = lambda i: (i, 0)
    whole = lambda i: (0, 0)
    eye = jnp.eye(k_nb, dtype=jnp.float32)
    w1f = jnp.kron(eye, W1[0:cin, :])
    w1p = jnp.kron(eye, W1[cin : cin + 3, :])
    w2w = jnp.kron(eye, W2)
    w3w = jnp.kron(eye, W3)
    bt = lambda v: jnp.tile(v.reshape(1, -1), (1, k_nb))
    vec = lambda v: v.reshape(1, -1)
    st_shape = jax.ShapeDtypeStruct((8, wide), jnp.float32)
    st_spec = pl.BlockSpec((8, wide), whole)

    y1, st1 = pl.pallas_call(
        functools.partial(_mlp1_body, m_rows),
        grid=grid,
        in_specs=[
            pl.BlockSpec((rn, wide), row2),
            pl.BlockSpec((rn, 3 * k_nb), row2),
            pl.BlockSpec(w1f.shape, whole),
            pl.BlockSpec(w1p.shape, whole),
            pl.BlockSpec((1, wide), whole),
        ],
        out_specs=[pl.BlockSpec((rn, wide), row2), st_spec],
        out_shape=[jax.ShapeDtypeStruct((npts, wide), jnp.float32), st_shape],
        interpret=interpret,
    )(fgw, pc48, w1f, w1p, bt(b1))

    y2, st2 = pl.pallas_call(
        functools.partial(_mlp2_body, m_rows, k_nb, cout),
        grid=grid,
        in_specs=[
            pl.BlockSpec((rn, wide), row2),
            st_spec,
            pl.BlockSpec(w2w.shape, whole),
            pl.BlockSpec((1, wide), whole),
            pl.BlockSpec((1, cout), whole),
            pl.BlockSpec((1, cout), whole),
        ],
        out_specs=[pl.BlockSpec((rn, wide), row2), st_spec],
        out_shape=[jax.ShapeDtypeStruct((npts, wide), jnp.float32), st_shape],
        interpret=interpret,
    )(y1, st1, w2w, bt(b2), vec(g1), vec(be1))

    out = pl.pallas_call(
        functools.partial(_mlp3_body, m_rows, k_nb, cout),
        grid=grid,
        in_specs=[
            pl.BlockSpec((rn, wide), row2),
            st_spec,
            pl.BlockSpec(w3w.shape, whole),
            pl.BlockSpec((1, wide), whole),
            pl.BlockSpec((1, cout), whole),
            pl.BlockSpec((1, cout), whole),
        ],
        out_specs=pl.BlockSpec((rn, cout), row2),
        out_shape=jax.ShapeDtypeStruct((npts, cout), jnp.float32),
        interpret=interpret,
    )(y2, st2, w3w, bt(b3), vec(g2), vec(be2))
    return out


def kernel(features, position_condition, W1, b1, g1, be1, W2, b2, g2, be2,
           W3, b3):
    b, n, cin = features.shape
    k_nb = position_condition.shape[2]
    m_rows = b * n * k_nb

    pc48 = position_condition.reshape(b, n, 3 * k_nb)
    pct = position_condition.transpose(0, 2, 3, 1).reshape(b, 3 * k_nb, n)
    p = _prep(pct)
    idx = _knn(p, k_nb, rb=256)
    table = features.reshape(b * n, cin)
    fg = _sc_gather(table, idx.reshape(m_rows // 128, 128))
    fgw = fg.reshape(b * n, k_nb * cin)
    out = _mlp(fgw, pc48.reshape(b * n, 3 * k_nb), k_nb,
               W1, b1, g1, be1, W2, b2, g2, be2, W3, b3, rn=512)
    return out.reshape(b, n, W3.shape[-1])


# speedup vs baseline: 1.2585x; 1.1087x over previous
import functools

import jax
import jax.numpy as jnp
from jax import lax
from jax.experimental import pallas as pl
from jax.experimental.pallas import tpu as pltpu
from jax.experimental.pallas import tpu_sc as plsc


def _prep_body(k_nb, pct_ref, p_ref):
    x = pct_ref[0]
    n = x.shape[-1]
    x = x.reshape(k_nb, 3, n)
    xyz = jnp.mean(x, axis=0)
    sq = jnp.sum(xyz * xyz, axis=0, keepdims=True)
    p_ref[0] = jnp.concatenate([xyz, sq], axis=0)


def _prep(pct, interpret=False):
    b, threek, n = pct.shape
    return pl.pallas_call(
        functools.partial(_prep_body, threek // 3),
        grid=(b,),
        in_specs=[pl.BlockSpec((1, threek, n), lambda i: (i, 0, 0))],
        out_specs=pl.BlockSpec((1, 4, n), lambda i: (i, 0, 0)),
        out_shape=jax.ShapeDtypeStruct((b, 4, n), jnp.float32),
        interpret=interpret,
    )(pct)


_ROUNDS = 4
_LANES = 128


def _knn_body(rb, k_nb, pfull_ref, prows_ref, idx_ref, d_ref, cv_ref, ci_ref):
    bi = pl.program_id(0)
    n = pfull_ref.shape[-1]
    nsl = n // _LANES
    cw = cv_ref.shape[-1]
    inf = jnp.float32(jnp.inf)
    xyzT = pfull_ref[0, 0:3, :]
    sqj = pfull_ref[0, 3:4, :]
    lhs = prows_ref[0, 0:3, :]
    sqi = prows_ref[0, 3:4, :]
    dn = (((0,), (0,)), ((), ()))
    dot = lax.dot_general(lhs, xyzT, dn, preferred_element_type=jnp.float32)
    ones = jnp.ones((1, n), jnp.float32)
    sqib = lax.dot_general(sqi, ones, dn, preferred_element_type=jnp.float32)
    d_ref[...] = (sqib + sqj) - 2.0 * dot
    cv_ref[...] = jnp.full((rb, cw), inf, jnp.float32)
    ci_ref[...] = jnp.full((rb, cw), jnp.int32(1 << 30), jnp.int32)

    lane = lax.broadcasted_iota(jnp.int32, (rb, _LANES), 1)

    def fold_min_argmin():
        u = d_ref[:, 0:_LANES]
        sidx = jnp.zeros((rb, _LANES), jnp.int32)
        for s in range(1, nsl):
            sl = d_ref[:, s * _LANES : (s + 1) * _LANES]
            take = sl < u
            u = jnp.where(take, sl, u)
            sidx = jnp.where(take, s, sidx)
        return u, sidx

    for r in range(_ROUNDS):
        u, sidx = fold_min_argmin()
        cv_ref[:, r * _LANES : (r + 1) * _LANES] = u
        ci_ref[:, r * _LANES : (r + 1) * _LANES] = sidx * _LANES + lane
        for s in range(nsl):
            sl = d_ref[:, s * _LANES : (s + 1) * _LANES]
            d_ref[:, s * _LANES : (s + 1) * _LANES] = jnp.where(
                sidx == s, inf, sl)

    u = d_ref[:, 0:_LANES]
    for s in range(1, nsl):
        u = jnp.minimum(u, d_ref[:, s * _LANES : (s + 1) * _LANES])
    tau = jnp.min(u, axis=1, keepdims=True)
    cnt = jnp.sum((cv_ref[...] < tau).astype(jnp.int32), axis=1,
                  keepdims=True)
    ok = jnp.min(cnt) >= k_nb

    @pl.when(jnp.logical_not(ok))
    def _fallback():
        iota = lax.broadcasted_iota(jnp.int32, (rb, n), 1)
        bign = jnp.int32(n)
        base = _ROUNDS * _LANES
        for k in range(k_nb):
            dd = d_ref[...]
            m = jnp.min(dd, axis=1, keepdims=True)
            t = jnp.where(dd == m, iota, bign)
            a = jnp.min(t, axis=1, keepdims=True)
            cv_ref[:, base + k : base + k + 1] = m
            ci_ref[:, base + k : base + k + 1] = a
            d_ref[...] = jnp.where(iota == a, inf, dd)

    cv = cv_ref[...]
    ci = ci_ref[...]
    bigi = jnp.int32(1 << 30)
    iota_k = lax.broadcasted_iota(jnp.int32, (rb, k_nb), 1)
    res = jnp.zeros((rb, k_nb), jnp.int32)
    for k in range(k_nb):
        m = jnp.min(cv, axis=1, keepdims=True)
        t = jnp.where(cv == m, ci, bigi)
        a = jnp.min(t, axis=1, keepdims=True)
        res = jnp.where(iota_k == k, a, res)
        cv = jnp.where(ci == a, inf, cv)
    idx_ref[0] = res + bi * n


def _knn(p, k_nb, rb, interpret=False):
    b, _, n = p.shape
    cw = (_ROUNDS + 1) * _LANES
    return pl.pallas_call(
        functools.partial(_knn_body, rb, k_nb),
        grid=(b, n // rb),
        in_specs=[
            pl.BlockSpec((1, 4, n), lambda i, j: (i, 0, 0)),
            pl.BlockSpec((1, 4, rb), lambda i, j: (i, 0, j)),
        ],
        out_specs=pl.BlockSpec((1, rb, k_nb), lambda i, j: (i, j, 0)),
        out_shape=jax.ShapeDtypeStruct((b, n, k_nb), jnp.int32),
        scratch_shapes=[
            pltpu.VMEM((rb, n), jnp.float32),
            pltpu.VMEM((rb, cw), jnp.float32),
            pltpu.VMEM((rb, cw), jnp.int32),
        ],
        interpret=interpret,
    )(p, p)


def _sc_gather(table, idx2d):
    info = plsc.get_sparse_core_info()
    nw = info.num_cores * info.num_subcores
    nrows_idx, lanes = idx2d.shape
    jpw = nrows_idx // nw
    cin = table.shape[-1]
    m_rows = nrows_idx * lanes
    mesh = plsc.VectorSubcoreMesh(core_axis_name="c", subcore_axis_name="s")

    @functools.partial(
        pl.kernel,
        mesh=mesh,
        compiler_params=pltpu.CompilerParams(use_tc_tiling_on_sc=False),
        out_type=jax.ShapeDtypeStruct((m_rows, cin), jnp.float32),
        scratch_types=[
            pltpu.VMEM((jpw, lanes), jnp.int32),
            pltpu.VMEM((lanes, cin), jnp.float32),
            pltpu.VMEM((lanes, cin), jnp.float32),
            pltpu.SemaphoreType.DMA,
            pltpu.SemaphoreType.DMA,
        ],
    )
    def k(table_hbm, idx_hbm, out_hbm, idx_all, rows0, rows1, sem0, sem1):
        wid = lax.axis_index("s") * info.num_cores + lax.axis_index("c")
        base = wid * jpw
        pltpu.sync_copy(idx_hbm.at[pl.ds(base, jpw)], idx_all)
        rows = (rows0, rows1)
        sems = (sem0, sem1)

        pltpu.async_copy(table_hbm.at[idx_all.at[0]], rows0, sem0)
        pltpu.async_copy(table_hbm.at[idx_all.at[1]], rows1, sem1)

        def body(i, _):
            for t in range(2):
                j = 2 * i + t
                pltpu.make_async_copy(
                    table_hbm.at[idx_all.at[j]], rows[t], sems[t]
                ).wait()
                pltpu.sync_copy(
                    rows[t], out_hbm.at[pl.ds((base + j) * lanes, lanes)]
                )

                @pl.when(j + 2 < jpw)
                def _():
                    pltpu.async_copy(
                        table_hbm.at[idx_all.at[j + 2]], rows[t], sems[t]
                    )

            return 0

        lax.fori_loop(0, jpw // 2, body, 0)

    return k(table, idx2d)


def _mlp1_body(m_rows, fgw_ref, pc_ref, w1f_ref, w1p_ref, b1_ref,
               y1_ref, st_ref):
    i = pl.program_id(0)
    y = (
        jnp.dot(fgw_ref[...], w1f_ref[...], preferred_element_type=jnp.float32)
        + jnp.dot(pc_ref[...], w1p_ref[...], preferred_element_type=jnp.float32)
        + b1_ref[...]
    )
    y1_ref[...] = y

    @pl.when(i == 0)
    def _():
        st_ref[...] = jnp.zeros_like(st_ref)

    st_ref[0:1, :] += jnp.sum(y, axis=0, keepdims=True)
    st_ref[1:2, :] += jnp.sum(y * y, axis=0, keepdims=True)


def _bn_affine(st_ref, g_ref, be_ref, m_rows, k_nb, cout):
    ssum = st_ref[0:1, 0:cout]
    ssq = st_ref[1:2, 0:cout]
    for k in range(1, k_nb):
        ssum = ssum + st_ref[0:1, k * cout : (k + 1) * cout]
        ssq = ssq + st_ref[1:2, k * cout : (k + 1) * cout]
    mm = ssum / m_rows
    vv = ssq / m_rows - mm * mm
    a = g_ref[...] / jnp.sqrt(vv + 1e-5)
    c = be_ref[...] - mm * a
    a = jnp.concatenate([a] * k_nb, axis=1)
    c = jnp.concatenate([c] * k_nb, axis=1)
    return a, c


def _mlp2_body(m_rows, k_nb, cout, y1_ref, st1_ref, w2_ref, b2_ref, g1_ref,
               be1_ref, y2_ref, st2_ref):
    i = pl.program_id(0)
    a, c = _bn_affine(st1_ref, g1_ref, be1_ref, m_rows, k_nb, cout)
    h = jnp.maximum(y1_ref[...] * a + c, 0.0)
    y = jnp.dot(h, w2_ref[...], preferred_element_type=jnp.float32) + b2_ref[...]
    y2_ref[...] = y

    @pl.when(i == 0)
    def _():
        st2_ref[...] = jnp.zeros_like(st2_ref)

    st2_ref[0:1, :] += jnp.sum(y, axis=0, keepdims=True)
    st2_ref[1:2, :] += jnp.sum(y * y, axis=0, keepdims=True)


def _mlp3_body(m_rows, k_nb, cout, y2_ref, st2_ref, w3_ref, b3_ref, g2_ref,
               be2_ref, out_ref):
    a, c = _bn_affine(st2_ref, g2_ref, be2_ref, m_rows, k_nb, cout)
    h = jnp.maximum(y2_ref[...] * a + c, 0.0)
    y = jnp.dot(h, w3_ref[...], preferred_element_type=jnp.float32) + b3_ref[...]
    r = y[:, 0:cout]
    for k in range(1, k_nb):
        r = jnp.maximum(r, y[:, k * cout : (k + 1) * cout])
    out_ref[...] = r


def _mlp(fgw, pc48, k_nb, W1, b1, g1, be1, W2, b2, g2, be2, W3, b3, rn,
         interpret=False):
    npts, wide = fgw.shape
    cout = W1.shape[-1]
    cin = W1.shape[0] - 3
    m_rows = float(npts * k_nb)
    grid = (npts // rn,)
    row2 = lambda i: (i, 0)
    whole = lambda i: (0, 0)
    eye = jnp.eye(k_nb, dtype=jnp.float32)
    w1f = jnp.kron(eye, W1[0:cin, :])
    w1p = jnp.kron(eye, W1[cin : cin + 3, :])
    w2w = jnp.kron(eye, W2)
    w3w = jnp.kron(eye, W3)
    bt = lambda v: jnp.tile(v.reshape(1, -1), (1, k_nb))
    vec = lambda v: v.reshape(1, -1)
    st_shape = jax.ShapeDtypeStruct((8, wide), jnp.float32)
    st_spec = pl.BlockSpec((8, wide), whole)

    y1, st1 = pl.pallas_call(
        functools.partial(_mlp1_body, m_rows),
        grid=grid,
        in_specs=[
            pl.BlockSpec((rn, wide), row2),
            pl.BlockSpec((rn, 3 * k_nb), row2),
            pl.BlockSpec(w1f.shape, whole),
            pl.BlockSpec(w1p.shape, whole),
            pl.BlockSpec((1, wide), whole),
        ],
        out_specs=[pl.BlockSpec((rn, wide), row2), st_spec],
        out_shape=[jax.ShapeDtypeStruct((npts, wide), jnp.float32), st_shape],
        interpret=interpret,
    )(fgw, pc48, w1f, w1p, bt(b1))

    y2, st2 = pl.pallas_call(
        functools.partial(_mlp2_body, m_rows, k_nb, cout),
        grid=grid,
        in_specs=[
            pl.BlockSpec((rn, wide), row2),
            st_spec,
            pl.BlockSpec(w2w.shape, whole),
            pl.BlockSpec((1, wide), whole),
            pl.BlockSpec((1, cout), whole),
            pl.BlockSpec((1, cout), whole),
        ],
        out_specs=[pl.BlockSpec((rn, wide), row2), st_spec],
        out_shape=[jax.ShapeDtypeStruct((npts, wide), jnp.float32), st_shape],
        interpret=interpret,
    )(y1, st1, w2w, bt(b2), vec(g1), vec(be1))

    out = pl.pallas_call(
        functools.partial(_mlp3_body, m_rows, k_nb, cout),
        grid=grid,
        in_specs=[
            pl.BlockSpec((rn, wide), row2),
            st_spec,
            pl.BlockSpec(w3w.shape, whole),
            pl.BlockSpec((1, wide), whole),
            pl.BlockSpec((1, cout), whole),
            pl.BlockSpec((1, cout), whole),
        ],
        out_specs=pl.BlockSpec((rn, cout), row2),
        out_shape=jax.ShapeDtypeStruct((npts, cout), jnp.float32),
        interpret=interpret,
    )(y2, st2, w3w, bt(b3), vec(g2), vec(be2))
    return out


def kernel(features, position_condition, W1, b1, g1, be1, W2, b2, g2, be2,
           W3, b3):
    b, n, cin = features.shape
    k_nb = position_condition.shape[2]
    m_rows = b * n * k_nb

    pc48 = position_condition.reshape(b, n, 3 * k_nb)
    pct = position_condition.transpose(0, 2, 3, 1).reshape(b, 3 * k_nb, n)
    p = _prep(pct)
    idx = _knn(p, k_nb, rb=256)
    table = features.reshape(b * n, cin)
    fg = _sc_gather(table, idx.reshape(m_rows // 128, 128))
    fgw = fg.reshape(b * n, k_nb * cin)
    out = _mlp(fgw, pc48.reshape(b * n, 3 * k_nb), k_nb,
               W1, b1, g1, be1, W2, b2, g2, be2, W3, b3, rn=512)
    return out.reshape(b, n, W3.shape[-1])


# speedup vs baseline: 1.2633x; 1.0038x over previous
import functools

import jax
import jax.numpy as jnp
from jax import lax
from jax.experimental import pallas as pl
from jax.experimental.pallas import tpu as pltpu
from jax.experimental.pallas import tpu_sc as plsc


def _prep_body(k_nb, pct_ref, p_ref):
    x = pct_ref[0]
    n = x.shape[-1]
    x = x.reshape(k_nb, 3, n)
    xyz = jnp.mean(x, axis=0)
    sq = jnp.sum(xyz * xyz, axis=0, keepdims=True)
    p_ref[0] = jnp.concatenate([xyz, sq], axis=0)


def _prep(pct, interpret=False):
    b, threek, n = pct.shape
    return pl.pallas_call(
        functools.partial(_prep_body, threek // 3),
        grid=(b,),
        in_specs=[pl.BlockSpec((1, threek, n), lambda i: (i, 0, 0))],
        out_specs=pl.BlockSpec((1, 4, n), lambda i: (i, 0, 0)),
        out_shape=jax.ShapeDtypeStruct((b, 4, n), jnp.float32),
        interpret=interpret,
    )(pct)


_ROUNDS = 4
_LANES = 128


def _knn_body(rb, k_nb, pfull_ref, prows_ref, idx_ref, d_ref, cv_ref, ci_ref):
    bi = pl.program_id(0)
    n = pfull_ref.shape[-1]
    nsl = n // _LANES
    cw = cv_ref.shape[-1]
    inf = jnp.float32(jnp.inf)
    xyzT = pfull_ref[0, 0:3, :]
    sqj = pfull_ref[0, 3:4, :]
    lhs = prows_ref[0, 0:3, :]
    sqi = prows_ref[0, 3:4, :]
    dn = (((0,), (0,)), ((), ()))
    dot = lax.dot_general(lhs, xyzT, dn, preferred_element_type=jnp.float32)
    ones = jnp.ones((1, n), jnp.float32)
    sqib = lax.dot_general(sqi, ones, dn, preferred_element_type=jnp.float32)
    d_ref[...] = (sqib + sqj) - 2.0 * dot
    cv_ref[...] = jnp.full((rb, cw), inf, jnp.float32)
    ci_ref[...] = jnp.full((rb, cw), jnp.int32(1 << 30), jnp.int32)

    lane = lax.broadcasted_iota(jnp.int32, (rb, _LANES), 1)

    def fold_min_argmin():
        u = d_ref[:, 0:_LANES]
        sidx = jnp.zeros((rb, _LANES), jnp.int32)
        for s in range(1, nsl):
            sl = d_ref[:, s * _LANES : (s + 1) * _LANES]
            take = sl < u
            u = jnp.where(take, sl, u)
            sidx = jnp.where(take, s, sidx)
        return u, sidx

    def round_body(r):
        u, sidx = fold_min_argmin()
        cv_ref[:, r * _LANES : (r + 1) * _LANES] = u
        ci_ref[:, r * _LANES : (r + 1) * _LANES] = sidx * _LANES + lane
        for s in range(nsl):
            sl = d_ref[:, s * _LANES : (s + 1) * _LANES]
            d_ref[:, s * _LANES : (s + 1) * _LANES] = jnp.where(
                sidx == s, inf, sl)

    def coverage_ok():
        u = d_ref[:, 0:_LANES]
        for s in range(1, nsl):
            u = jnp.minimum(u, d_ref[:, s * _LANES : (s + 1) * _LANES])
        tau = jnp.min(u, axis=1, keepdims=True)
        cnt = jnp.sum((cv_ref[...] < tau).astype(jnp.int32), axis=1,
                      keepdims=True)
        return jnp.min(cnt) >= k_nb

    for r in range(_ROUNDS - 1):
        round_body(r)

    @pl.when(jnp.logical_not(coverage_ok()))
    def _round_last():
        round_body(_ROUNDS - 1)

    @pl.when(jnp.logical_not(coverage_ok()))
    def _fallback():
        iota = lax.broadcasted_iota(jnp.int32, (rb, n), 1)
        bign = jnp.int32(n)
        base = _ROUNDS * _LANES
        for k in range(k_nb):
            dd = d_ref[...]
            m = jnp.min(dd, axis=1, keepdims=True)
            t = jnp.where(dd == m, iota, bign)
            a = jnp.min(t, axis=1, keepdims=True)
            cv_ref[:, base + k : base + k + 1] = m
            ci_ref[:, base + k : base + k + 1] = a
            d_ref[...] = jnp.where(iota == a, inf, dd)

    cv = cv_ref[...]
    ci = ci_ref[...]
    bigi = jnp.int32(1 << 30)
    iota_k = lax.broadcasted_iota(jnp.int32, (rb, k_nb), 1)
    res = jnp.zeros((rb, k_nb), jnp.int32)
    for k in range(k_nb):
        m = jnp.min(cv, axis=1, keepdims=True)
        t = jnp.where(cv == m, ci, bigi)
        a = jnp.min(t, axis=1, keepdims=True)
        res = jnp.where(iota_k == k, a, res)
        cv = jnp.where(ci == a, inf, cv)
    idx_ref[0] = res + bi * n


def _knn(p, k_nb, rb, interpret=False):
    b, _, n = p.shape
    cw = (_ROUNDS + 1) * _LANES
    return pl.pallas_call(
        functools.partial(_knn_body, rb, k_nb),
        grid=(b, n // rb),
        in_specs=[
            pl.BlockSpec((1, 4, n), lambda i, j: (i, 0, 0)),
            pl.BlockSpec((1, 4, rb), lambda i, j: (i, 0, j)),
        ],
        out_specs=pl.BlockSpec((1, rb, k_nb), lambda i, j: (i, j, 0)),
        out_shape=jax.ShapeDtypeStruct((b, n, k_nb), jnp.int32),
        scratch_shapes=[
            pltpu.VMEM((rb, n), jnp.float32),
            pltpu.VMEM((rb, cw), jnp.float32),
            pltpu.VMEM((rb, cw), jnp.int32),
        ],
        interpret=interpret,
    )(p, p)


def _sc_gather(table, idx2d):
    info = plsc.get_sparse_core_info()
    nw = info.num_cores * info.num_subcores
    nrows_idx, lanes = idx2d.shape
    jpw = nrows_idx // nw
    cin = table.shape[-1]
    m_rows = nrows_idx * lanes
    mesh = plsc.VectorSubcoreMesh(core_axis_name="c", subcore_axis_name="s")

    @functools.partial(
        pl.kernel,
        mesh=mesh,
        compiler_params=pltpu.CompilerParams(use_tc_tiling_on_sc=False),
        out_type=jax.ShapeDtypeStruct((m_rows, cin), jnp.float32),
        scratch_types=[
            pltpu.VMEM((jpw, lanes), jnp.int32),
            pltpu.VMEM((lanes, cin), jnp.float32),
            pltpu.VMEM((lanes, cin), jnp.float32),
            pltpu.SemaphoreType.DMA,
            pltpu.SemaphoreType.DMA,
        ],
    )
    def k(table_hbm, idx_hbm, out_hbm, idx_all, rows0, rows1, sem0, sem1):
        wid = lax.axis_index("s") * info.num_cores + lax.axis_index("c")
        base = wid * jpw
        pltpu.sync_copy(idx_hbm.at[pl.ds(base, jpw)], idx_all)
        rows = (rows0, rows1)
        sems = (sem0, sem1)

        pltpu.async_copy(table_hbm.at[idx_all.at[0]], rows0, sem0)
        pltpu.async_copy(table_hbm.at[idx_all.at[1]], rows1, sem1)

        def body(i, _):
            for t in range(2):
                j = 2 * i + t
                pltpu.make_async_copy(
                    table_hbm.at[idx_all.at[j]], rows[t], sems[t]
                ).wait()
                pltpu.sync_copy(
                    rows[t], out_hbm.at[pl.ds((base + j) * lanes, lanes)]
                )

                @pl.when(j + 2 < jpw)
                def _():
                    pltpu.async_copy(
                        table_hbm.at[idx_all.at[j + 2]], rows[t], sems[t]
                    )

            return 0

        lax.fori_loop(0, jpw // 2, body, 0)

    return k(table, idx2d)


def _mlp1_body(m_rows, fgw_ref, pc_ref, w1f_ref, w1p_ref, b1_ref,
               y1_ref, st_ref):
    i = pl.program_id(0)
    y = (
        jnp.dot(fgw_ref[...], w1f_ref[...], preferred_element_type=jnp.float32)
        + jnp.dot(pc_ref[...], w1p_ref[...], preferred_element_type=jnp.float32)
        + b1_ref[...]
    )
    y1_ref[...] = y

    @pl.when(i == 0)
    def _():
        st_ref[...] = jnp.zeros_like(st_ref)

    st_ref[0:1, :] += jnp.sum(y, axis=0, keepdims=True)
    st_ref[1:2, :] += jnp.sum(y * y, axis=0, keepdims=True)


def _bn_affine(st_ref, g_ref, be_ref, m_rows, k_nb, cout):
    ssum = st_ref[0:1, 0:cout]
    ssq = st_ref[1:2, 0:cout]
    for k in range(1, k_nb):
        ssum = ssum + st_ref[0:1, k * cout : (k + 1) * cout]
        ssq = ssq + st_ref[1:2, k * cout : (k + 1) * cout]
    mm = ssum / m_rows
    vv = ssq / m_rows - mm * mm
    a = g_ref[...] / jnp.sqrt(vv + 1e-5)
    c = be_ref[...] - mm * a
    a = jnp.concatenate([a] * k_nb, axis=1)
    c = jnp.concatenate([c] * k_nb, axis=1)
    return a, c


def _mlp2_body(m_rows, k_nb, cout, y1_ref, st1_ref, w2_ref, b2_ref, g1_ref,
               be1_ref, y2_ref, st2_ref):
    i = pl.program_id(0)
    a, c = _bn_affine(st1_ref, g1_ref, be1_ref, m_rows, k_nb, cout)
    h = jnp.maximum(y1_ref[...] * a + c, 0.0)
    y = jnp.dot(h, w2_ref[...], preferred_element_type=jnp.float32) + b2_ref[...]
    y2_ref[...] = y

    @pl.when(i == 0)
    def _():
        st2_ref[...] = jnp.zeros_like(st2_ref)

    st2_ref[0:1, :] += jnp.sum(y, axis=0, keepdims=True)
    st2_ref[1:2, :] += jnp.sum(y * y, axis=0, keepdims=True)


def _mlp3_body(m_rows, k_nb, cout, y2_ref, st2_ref, w3_ref, b3_ref, g2_ref,
               be2_ref, out_ref):
    a, c = _bn_affine(st2_ref, g2_ref, be2_ref, m_rows, k_nb, cout)
    h = jnp.maximum(y2_ref[...] * a + c, 0.0)
    y = jnp.dot(h, w3_ref[...], preferred_element_type=jnp.float32) + b3_ref[...]
    r = y[:, 0:cout]
    for k in range(1, k_nb):
        r = jnp.maximum(r, y[:, k * cout : (k + 1) * cout])
    out_ref[...] = r


def _mlp(fgw, pc48, k_nb, W1, b1, g1, be1, W2, b2, g2, be2, W3, b3, rn,
         interpret=False):
    npts, wide = fgw.shape
    cout = W1.shape[-1]
    cin = W1.shape[0] - 3
    m_rows = float(npts * k_nb)
    grid = (npts // rn,)
    row2 = lambda i: (i, 0)
    whole = lambda i: (0, 0)
    eye = jnp.eye(k_nb, dtype=jnp.float32)
    w1f = jnp.kron(eye, W1[0:cin, :])
    w1p = jnp.kron(eye, W1[cin : cin + 3, :])
    w2w = jnp.kron(eye, W2)
    w3w = jnp.kron(eye, W3)
    bt = lambda v: jnp.tile(v.reshape(1, -1), (1, k_nb))
    vec = lambda v: v.reshape(1, -1)
    st_shape = jax.ShapeDtypeStruct((8, wide), jnp.float32)
    st_spec = pl.BlockSpec((8, wide), whole)

    y1, st1 = pl.pallas_call(
        functools.partial(_mlp1_body, m_rows),
        grid=grid,
        in_specs=[
            pl.BlockSpec((rn, wide), row2),
            pl.BlockSpec((rn, 3 * k_nb), row2),
            pl.BlockSpec(w1f.shape, whole),
            pl.BlockSpec(w1p.shape, whole),
            pl.BlockSpec((1, wide), whole),
        ],
        out_specs=[pl.BlockSpec((rn, wide), row2), st_spec],
        out_shape=[jax.ShapeDtypeStruct((npts, wide), jnp.float32), st_shape],
        interpret=interpret,
    )(fgw, pc48, w1f, w1p, bt(b1))

    y2, st2 = pl.pallas_call(
        functools.partial(_mlp2_body, m_rows, k_nb, cout),
        grid=grid,
        in_specs=[
            pl.BlockSpec((rn, wide), row2),
            st_spec,
            pl.BlockSpec(w2w.shape, whole),
            pl.BlockSpec((1, wide), whole),
            pl.BlockSpec((1, cout), whole),
            pl.BlockSpec((1, cout), whole),
        ],
        out_specs=[pl.BlockSpec((rn, wide), row2), st_spec],
        out_shape=[jax.ShapeDtypeStruct((npts, wide), jnp.float32), st_shape],
        interpret=interpret,
    )(y1, st1, w2w, bt(b2), vec(g1), vec(be1))

    out = pl.pallas_call(
        functools.partial(_mlp3_body, m_rows, k_nb, cout),
        grid=grid,
        in_specs=[
            pl.BlockSpec((rn, wide), row2),
            st_spec,
            pl.BlockSpec(w3w.shape, whole),
            pl.BlockSpec((1, wide), whole),
            pl.BlockSpec((1, cout), whole),
            pl.BlockSpec((1, cout), whole),
        ],
        out_specs=pl.BlockSpec((rn, cout), row2),
        out_shape=jax.ShapeDtypeStruct((npts, cout), jnp.float32),
        interpret=interpret,
    )(y2, st2, w3w, bt(b3), vec(g2), vec(be2))
    return out


def kernel(features, position_condition, W1, b1, g1, be1, W2, b2, g2, be2,
           W3, b3):
    b, n, cin = features.shape
    k_nb = position_condition.shape[2]
    m_rows = b * n * k_nb

    pc48 = position_condition.reshape(b, n, 3 * k_nb)
    pct = position_condition.transpose(0, 2, 3, 1).reshape(b, 3 * k_nb, n)
    p = _prep(pct)
    idx = _knn(p, k_nb, rb=256)
    table = features.reshape(b * n, cin)
    fg = _sc_gather(table, idx.reshape(m_rows // 128, 128))
    fgw = fg.reshape(b * n, k_nb * cin)
    out = _mlp(fgw, pc48.reshape(b * n, 3 * k_nb), k_nb,
               W1, b1, g1, be1, W2, b2, g2, be2, W3, b3, rn=512)
    return out.reshape(b, n, W3.shape[-1])


# speedup vs baseline: 1.2699x; 1.0053x over previous
import functools

import jax
import jax.numpy as jnp
from jax import lax
from jax.experimental import pallas as pl
from jax.experimental.pallas import tpu as pltpu
from jax.experimental.pallas import tpu_sc as plsc


def _prep_body(k_nb, pct_ref, p_ref):
    x = pct_ref[0]
    n = x.shape[-1]
    x = x.reshape(k_nb, 3, n)
    xyz = jnp.mean(x, axis=0)
    sq = jnp.sum(xyz * xyz, axis=0, keepdims=True)
    p_ref[0] = jnp.concatenate([xyz, sq], axis=0)


def _prep(pct, interpret=False):
    b, threek, n = pct.shape
    return pl.pallas_call(
        functools.partial(_prep_body, threek // 3),
        grid=(b,),
        in_specs=[pl.BlockSpec((1, threek, n), lambda i: (i, 0, 0))],
        out_specs=pl.BlockSpec((1, 4, n), lambda i: (i, 0, 0)),
        out_shape=jax.ShapeDtypeStruct((b, 4, n), jnp.float32),
        interpret=interpret,
    )(pct)


_ROUNDS = 4
_LANES = 128


def _knn_body(rb, k_nb, pfull_ref, prows_ref, idx_ref, d_ref, cv_ref, ci_ref):
    bi = pl.program_id(0)
    n = pfull_ref.shape[-1]
    nsl = n // _LANES
    cw = cv_ref.shape[-1]
    inf = jnp.float32(jnp.inf)
    xyzT = pfull_ref[0, 0:3, :]
    sqj = pfull_ref[0, 3:4, :]
    lhs = prows_ref[0, 0:3, :]
    sqi = prows_ref[0, 3:4, :]
    dn = (((0,), (0,)), ((), ()))
    dot = lax.dot_general(lhs, xyzT, dn, preferred_element_type=jnp.float32)
    ones = jnp.ones((1, n), jnp.float32)
    sqib = lax.dot_general(sqi, ones, dn, preferred_element_type=jnp.float32)
    d_ref[...] = (sqib + sqj) - 2.0 * dot
    cv_ref[...] = jnp.full((rb, cw), inf, jnp.float32)
    ci_ref[...] = jnp.full((rb, cw), jnp.int32(1 << 30), jnp.int32)

    lane = lax.broadcasted_iota(jnp.int32, (rb, _LANES), 1)

    def fold_min_argmin():
        u = d_ref[:, 0:_LANES]
        sidx = jnp.zeros((rb, _LANES), jnp.int32)
        for s in range(1, nsl):
            sl = d_ref[:, s * _LANES : (s + 1) * _LANES]
            take = sl < u
            u = jnp.where(take, sl, u)
            sidx = jnp.where(take, s, sidx)
        return u, sidx

    def round_body(r):
        u, sidx = fold_min_argmin()
        cv_ref[:, r * _LANES : (r + 1) * _LANES] = u
        ci_ref[:, r * _LANES : (r + 1) * _LANES] = sidx * _LANES + lane
        for s in range(nsl):
            sl = d_ref[:, s * _LANES : (s + 1) * _LANES]
            d_ref[:, s * _LANES : (s + 1) * _LANES] = jnp.where(
                sidx == s, inf, sl)

    def coverage_ok():
        u = d_ref[:, 0:_LANES]
        for s in range(1, nsl):
            u = jnp.minimum(u, d_ref[:, s * _LANES : (s + 1) * _LANES])
        tau = jnp.min(u, axis=1, keepdims=True)
        cnt = jnp.sum((cv_ref[...] < tau).astype(jnp.int32), axis=1,
                      keepdims=True)
        return jnp.min(cnt) >= k_nb

    for r in range(_ROUNDS - 1):
        round_body(r)

    @pl.when(jnp.logical_not(coverage_ok()))
    def _round_last():
        round_body(_ROUNDS - 1)

    @pl.when(jnp.logical_not(coverage_ok()))
    def _fallback():
        iota = lax.broadcasted_iota(jnp.int32, (rb, n), 1)
        bign = jnp.int32(n)
        base = _ROUNDS * _LANES
        for k in range(k_nb):
            dd = d_ref[...]
            m = jnp.min(dd, axis=1, keepdims=True)
            t = jnp.where(dd == m, iota, bign)
            a = jnp.min(t, axis=1, keepdims=True)
            cv_ref[:, base + k : base + k + 1] = m
            ci_ref[:, base + k : base + k + 1] = a
            d_ref[...] = jnp.where(iota == a, inf, dd)

    cv = cv_ref[...]
    ci = ci_ref[...]
    bigi = jnp.int32(1 << 30)
    iota_k = lax.broadcasted_iota(jnp.int32, (rb, k_nb), 1)
    res = jnp.zeros((rb, k_nb), jnp.int32)
    for k in range(k_nb):
        m = jnp.min(cv, axis=1, keepdims=True)
        t = jnp.where(cv == m, ci, bigi)
        a = jnp.min(t, axis=1, keepdims=True)
        res = jnp.where(iota_k == k, a, res)
        cv = jnp.where(ci == a, inf, cv)
    idx_ref[0] = res + bi * n


def _knn(p, k_nb, rb, interpret=False):
    b, _, n = p.shape
    cw = (_ROUNDS + 1) * _LANES
    return pl.pallas_call(
        functools.partial(_knn_body, rb, k_nb),
        grid=(b, n // rb),
        in_specs=[
            pl.BlockSpec((1, 4, n), lambda i, j: (i, 0, 0)),
            pl.BlockSpec((1, 4, rb), lambda i, j: (i, 0, j)),
        ],
        out_specs=pl.BlockSpec((1, rb, k_nb), lambda i, j: (i, j, 0)),
        out_shape=jax.ShapeDtypeStruct((b, n, k_nb), jnp.int32),
        scratch_shapes=[
            pltpu.VMEM((rb, n), jnp.float32),
            pltpu.VMEM((rb, cw), jnp.float32),
            pltpu.VMEM((rb, cw), jnp.int32),
        ],
        interpret=interpret,
    )(p, p)


def _sc_gather(table, idx2d):
    info = plsc.get_sparse_core_info()
    nw = info.num_cores * info.num_subcores
    nrows_idx, lanes = idx2d.shape
    jpw = nrows_idx // nw
    cin = table.shape[-1]
    m_rows = nrows_idx * lanes
    mesh = plsc.VectorSubcoreMesh(core_axis_name="c", subcore_axis_name="s")

    @functools.partial(
        pl.kernel,
        mesh=mesh,
        compiler_params=pltpu.CompilerParams(use_tc_tiling_on_sc=False),
        out_type=jax.ShapeDtypeStruct((m_rows, cin), jnp.float32),
        scratch_types=[
            pltpu.VMEM((jpw, lanes), jnp.int32),
            pltpu.VMEM((lanes, cin), jnp.float32),
            pltpu.VMEM((lanes, cin), jnp.float32),
            pltpu.VMEM((lanes, cin), jnp.float32),
            pltpu.VMEM((lanes, cin), jnp.float32),
            pltpu.SemaphoreType.DMA,
            pltpu.SemaphoreType.DMA,
            pltpu.SemaphoreType.DMA,
            pltpu.SemaphoreType.DMA,
        ],
    )
    def k(table_hbm, idx_hbm, out_hbm, idx_all, rows0, rows1, rows2, rows3,
          sem0, sem1, sem2, sem3):
        wid = lax.axis_index("s") * info.num_cores + lax.axis_index("c")
        base = wid * jpw
        pltpu.sync_copy(idx_hbm.at[pl.ds(base, jpw)], idx_all)
        rows = (rows0, rows1, rows2, rows3)
        sems = (sem0, sem1, sem2, sem3)
        nbuf = 4

        for t in range(nbuf):
            pltpu.async_copy(table_hbm.at[idx_all.at[t]], rows[t], sems[t])

        def body(i, _):
            for t in range(nbuf):
                j = nbuf * i + t
                pltpu.make_async_copy(
                    table_hbm.at[idx_all.at[j]], rows[t], sems[t]
                ).wait()
                pltpu.sync_copy(
                    rows[t], out_hbm.at[pl.ds((base + j) * lanes, lanes)]
                )

                @pl.when(j + nbuf < jpw)
                def _():
                    pltpu.async_copy(
                        table_hbm.at[idx_all.at[j + nbuf]], rows[t], sems[t]
                    )

            return 0

        lax.fori_loop(0, jpw // nbuf, body, 0)

    return k(table, idx2d)


def _mlp1_body(m_rows, fgw_ref, pc_ref, w1f_ref, w1p_ref, b1_ref,
               y1_ref, st_ref):
    i = pl.program_id(0)
    y = (
        jnp.dot(fgw_ref[...], w1f_ref[...], preferred_element_type=jnp.float32)
        + jnp.dot(pc_ref[...], w1p_ref[...], preferred_element_type=jnp.float32)
        + b1_ref[...]
    )
    y1_ref[...] = y

    @pl.when(i == 0)
    def _():
        st_ref[...] = jnp.zeros_like(st_ref)

    st_ref[0:1, :] += jnp.sum(y, axis=0, keepdims=True)
    st_ref[1:2, :] += jnp.sum(y * y, axis=0, keepdims=True)


def _bn_affine(st_ref, g_ref, be_ref, m_rows, k_nb, cout):
    ssum = st_ref[0:1, 0:cout]
    ssq = st_ref[1:2, 0:cout]
    for k in range(1, k_nb):
        ssum = ssum + st_ref[0:1, k * cout : (k + 1) * cout]
        ssq = ssq + st_ref[1:2, k * cout : (k + 1) * cout]
    mm = ssum / m_rows
    vv = ssq / m_rows - mm * mm
    a = g_ref[...] / jnp.sqrt(vv + 1e-5)
    c = be_ref[...] - mm * a
    a = jnp.concatenate([a] * k_nb, axis=1)
    c = jnp.concatenate([c] * k_nb, axis=1)
    return a, c


def _mlp2_body(m_rows, k_nb, cout, y1_ref, st1_ref, w2_ref, b2_ref, g1_ref,
               be1_ref, y2_ref, st2_ref):
    i = pl.program_id(0)
    a, c = _bn_affine(st1_ref, g1_ref, be1_ref, m_rows, k_nb, cout)
    h = jnp.maximum(y1_ref[...] * a + c, 0.0)
    y = jnp.dot(h, w2_ref[...], preferred_element_type=jnp.float32) + b2_ref[...]
    y2_ref[...] = y

    @pl.when(i == 0)
    def _():
        st2_ref[...] = jnp.zeros_like(st2_ref)

    st2_ref[0:1, :] += jnp.sum(y, axis=0, keepdims=True)
    st2_ref[1:2, :] += jnp.sum(y * y, axis=0, keepdims=True)


def _mlp3_body(m_rows, k_nb, cout, y2_ref, st2_ref, w3_ref, b3_ref, g2_ref,
               be2_ref, out_ref):
    a, c = _bn_affine(st2_ref, g2_ref, be2_ref, m_rows, k_nb, cout)
    h = jnp.maximum(y2_ref[...] * a + c, 0.0)
    y = jnp.dot(h, w3_ref[...], preferred_element_type=jnp.float32) + b3_ref[...]
    r = y[:, 0:cout]
    for k in range(1, k_nb):
        r = jnp.maximum(r, y[:, k * cout : (k + 1) * cout])
    out_ref[...] = r


def _mlp(fgw, pc48, k_nb, W1, b1, g1, be1, W2, b2, g2, be2, W3, b3, rn,
         interpret=False):
    npts, wide = fgw.shape
    cout = W1.shape[-1]
    cin = W1.shape[0] - 3
    m_rows = float(npts * k_nb)
    grid = (npts // rn,)
    row2 = lambda i: (i, 0)
    whole = lambda i: (0, 0)
    eye = jnp.eye(k_nb, dtype=jnp.float32)
    w1f = jnp.kron(eye, W1[0:cin, :])
    w1p = jnp.kron(eye, W1[cin : cin + 3, :])
    w2w = jnp.kron(eye, W2)
    w3w = jnp.kron(eye, W3)
    bt = lambda v: jnp.tile(v.reshape(1, -1), (1, k_nb))
    vec = lambda v: v.reshape(1, -1)
    st_shape = jax.ShapeDtypeStruct((8, wide), jnp.float32)
    st_spec = pl.BlockSpec((8, wide), whole)

    y1, st1 = pl.pallas_call(
        functools.partial(_mlp1_body, m_rows),
        grid=grid,
        in_specs=[
            pl.BlockSpec((rn, wide), row2),
            pl.BlockSpec((rn, 3 * k_nb), row2),
            pl.BlockSpec(w1f.shape, whole),
            pl.BlockSpec(w1p.shape, whole),
            pl.BlockSpec((1, wide), whole),
        ],
        out_specs=[pl.BlockSpec((rn, wide), row2), st_spec],
        out_shape=[jax.ShapeDtypeStruct((npts, wide), jnp.float32), st_shape],
        interpret=interpret,
    )(fgw, pc48, w1f, w1p, bt(b1))

    y2, st2 = pl.pallas_call(
        functools.partial(_mlp2_body, m_rows, k_nb, cout),
        grid=grid,
        in_specs=[
            pl.BlockSpec((rn, wide), row2),
            st_spec,
            pl.BlockSpec(w2w.shape, whole),
            pl.BlockSpec((1, wide), whole),
            pl.BlockSpec((1, cout), whole),
            pl.BlockSpec((1, cout), whole),
        ],
        out_specs=[pl.BlockSpec((rn, wide), row2), st_spec],
        out_shape=[jax.ShapeDtypeStruct((npts, wide), jnp.float32), st_shape],
        interpret=interpret,
    )(y1, st1, w2w, bt(b2), vec(g1), vec(be1))

    out = pl.pallas_call(
        functools.partial(_mlp3_body, m_rows, k_nb, cout),
        grid=grid,
        in_specs=[
            pl.BlockSpec((rn, wide), row2),
            st_spec,
            pl.BlockSpec(w3w.shape, whole),
            pl.BlockSpec((1, wide), whole),
            pl.BlockSpec((1, cout), whole),
            pl.BlockSpec((1, cout), whole),
        ],
        out_specs=pl.BlockSpec((rn, cout), row2),
        out_shape=jax.ShapeDtypeStruct((npts, cout), jnp.float32),
        interpret=interpret,
    )(y2, st2, w3w, bt(b3), vec(g2), vec(be2))
    return out


def kernel(features, position_condition, W1, b1, g1, be1, W2, b2, g2, be2,
           W3, b3):
    b, n, cin = features.shape
    k_nb = position_condition.shape[2]
    m_rows = b * n * k_nb

    pc48 = position_condition.reshape(b, n, 3 * k_nb)
    pct = position_condition.transpose(0, 2, 3, 1).reshape(b, 3 * k_nb, n)
    p = _prep(pct)
    idx = _knn(p, k_nb, rb=256)
    table = features.reshape(b * n, cin)
    fg = _sc_gather(table, idx.reshape(m_rows // 128, 128))
    fgw = fg.reshape(b * n, k_nb * cin)
    out = _mlp(fgw, pc48.reshape(b * n, 3 * k_nb), k_nb,
               W1, b1, g1, be1, W2, b2, g2, be2, W3, b3, rn=512)
    return out.reshape(b, n, W3.shape[-1])
